# baseline (speedup 1.0000x reference)
import jax
import jax.numpy as jnp
import numpy as np
from jax.experimental import pallas as pl
from jax.experimental.pallas import tpu as pltpu

BT = 128


def _net_kernel(x_ref, t1_ref, b1_ref, t2_ref, b2_ref,
                fw1_ref, fb1_ref, fw2_ref, fb2_ref, out_ref):
    xb = x_ref[...].astype(jnp.bfloat16)
    xcat = jnp.concatenate(
        [xb[:, 0:32, :], xb[:, 1:33, :], xb[:, 2:34, :]], axis=2)
    h1 = jnp.dot(xcat.reshape(BT * 32, 96), t1_ref[...],
                 preferred_element_type=jnp.float32)
    x1 = jnp.maximum(h1 + b1_ref[...], 0.0).astype(jnp.bfloat16)
    x1v = x1.reshape(BT, 32, 1536)

    pools = []
    for g in range(6):
        acc = None
        for d in range(3):
            xs = x1v[:, d:d + 24, 256 * g:256 * g + 256].reshape(BT * 24, 256)
            part = jnp.dot(xs, t2_ref[d], preferred_element_type=jnp.float32)
            acc = part if acc is None else acc + part
        h2 = jnp.maximum(acc + b2_ref[...], 0.0)
        m = jnp.max(h2.reshape(BT, 12, 2, 256), axis=2)
        a = m.reshape(BT, 12, 2, 128)
        pools.append(jnp.maximum(a[..., :64], a[..., 64:]))
    flat = jnp.concatenate(pools, axis=2)
    flat = flat.reshape(BT, 9216).astype(jnp.bfloat16)

    f1 = jnp.dot(flat, fw1_ref[...], preferred_element_type=jnp.float32)
    f1 = jnp.maximum(f1 + fb1_ref[...], 0.0).astype(jnp.bfloat16)
    f2 = jnp.dot(f1, fw2_ref[...], preferred_element_type=jnp.float32)
    out_ref[...] = (f2 + fb2_ref[...])[:, :10]


_J = np.arange(26)
_S = np.zeros((26, 48), np.float32)
for _g in range(6):
    for _s in range(6):
        _S[4 * _g + _s, 8 * _g + _s] = 1.0


def _prep(conv1_w, conv1_b, conv2_w, conv2_b, fc1_w, fc1_b, fc2_w, fc2_b):
    E1 = np.zeros((3, 32, 26), np.float32)
    for e in range(3):
        E1[e, _J + e, _J] = 1.0
    w1r = conv1_w[:, 0, :, :]
    t1 = jnp.einsum('epj,js,cde->dpsc', E1, _S, w1r).reshape(96, 1536)
    b1 = jnp.tile(jnp.pad(jnp.tile(conv1_b, 6).reshape(6, 32),
                          ((0, 2), (0, 0))).reshape(-1), 6).reshape(1, 1536)
    E2 = np.zeros((3, 8, 4), np.float32)
    for e in range(3):
        E2[e, np.arange(4) + e, np.arange(4)] = 1.0
    t2 = jnp.einsum('esq,oide->dsiqo', E2, conv2_w).reshape(3, 256, 256)
    b2 = jnp.tile(conv2_b, 4).reshape(1, 256)
    fw1 = fc1_w.reshape(128, 64, 12, 12).transpose(0, 2, 3, 1)
    fw1 = fw1.reshape(128, 9216).T
    fw2 = jnp.zeros((128, 128), jnp.float32).at[:, :10].set(fc2_w.T)
    fb2 = jnp.zeros((1, 128), jnp.float32).at[0, :10].set(fc2_b)
    return (t1.astype(jnp.bfloat16), b1, t2.astype(jnp.bfloat16), b2,
            fw1.astype(jnp.bfloat16), fc1_b.reshape(1, 128),
            fw2.astype(jnp.bfloat16), fb2)


def _call(xp, args, interpret=False):
    b = xp.shape[0]
    grid = b // BT
    const = lambda *shape: pl.BlockSpec(shape, lambda i: (0,) * len(shape))
    return pl.pallas_call(
        _net_kernel,
        grid=(grid,),
        in_specs=[
            pl.BlockSpec((BT, 40, 32), lambda i: (i, 0, 0)),
            const(96, 1536), const(1, 1536), const(3, 256, 256),
            const(1, 256), const(9216, 128), const(1, 128),
            const(128, 128), const(1, 128),
        ],
        out_specs=pl.BlockSpec((BT, 10), lambda i: (i, 0)),
        out_shape=jax.ShapeDtypeStruct((b, 10), jnp.float32),
        interpret=interpret,
    )(xp, *args)


def kernel(x, conv1_w, conv1_b, conv2_w, conv2_b, fc1_w, fc1_b, fc2_w, fc2_b):
    args = _prep(conv1_w, conv1_b, conv2_w, conv2_b,
                 fc1_w, fc1_b, fc2_w, fc2_b)
    xp = jnp.pad(x.reshape(x.shape[0], 28, 28), ((0, 0), (0, 12), (0, 4)))
    return _call(xp, args)


# speedup vs baseline: 2.3538x; 2.3538x over previous
import jax
import jax.numpy as jnp
import numpy as np
from jax.experimental import pallas as pl
from jax.experimental.pallas import tpu as pltpu

BT = 128


def _net_kernel(xp_ref, t1_ref, t2_ref, fw1_ref, fb1_ref, fw2_ref, fb2_ref,
                out_ref):
    x2d = xp_ref[...].reshape(40 * BT, 32)
    xcat = jnp.concatenate(
        [x2d[0:32 * BT], x2d[BT:33 * BT], x2d[2 * BT:34 * BT]], axis=1)
    h1 = jnp.dot(xcat, t1_ref[...], preferred_element_type=jnp.float32)
    x1 = jnp.maximum(h1.astype(jnp.bfloat16), 0)

    parts = []
    for g in range(6):
        acc = None
        for d in range(3):
            xs = x1[d * BT:(d + 24) * BT, 256 * g:256 * g + 256]
            p = jnp.dot(xs, t2_ref[d], preferred_element_type=jnp.float32)
            acc = p if acc is None else acc + p
        h2 = jnp.maximum(acc, 0.0)
        m = jnp.max(h2.reshape(12, 2, BT, 256), axis=1)
        p1 = jnp.maximum(m[..., 0:64], m[..., 64:128])
        p2 = jnp.maximum(m[..., 128:192], m[..., 192:256])
        parts.append(jnp.concatenate([p1, p2], axis=2).astype(jnp.bfloat16))
    flat = jnp.concatenate(parts, axis=2)
    flat = flat.reshape(12 * BT, 768)

    facc = None
    for t in range(12):
        ft = jnp.dot(flat[t * BT:(t + 1) * BT], fw1_ref[t],
                     preferred_element_type=jnp.float32)
        facc = ft if facc is None else facc + ft
    f1 = jnp.maximum(facc + fb1_ref[...], 0.0).astype(jnp.bfloat16)
    f2 = jnp.dot(f1, fw2_ref[...], preferred_element_type=jnp.float32)
    out_ref[...] = (f2 + fb2_ref[...])[:, :10]


_SEL1 = np.zeros((32, 3, 6, 8), np.float32)
_SEL2 = np.zeros((8, 3, 4), np.float32)
_B1 = np.zeros((3, 32), np.float32)
_ONE6 = np.zeros((6, 8), np.float32)
_ONESLOT = np.zeros((3, 32, 6, 8, 32), np.float32)
_S6 = np.zeros((8, 32), np.float32)
for _g in range(6):
    for _s in range(6):
        for _e in range(3):
            _SEL1[4 * _g + _s + _e, _e, _g, _s] = 1.0
    _ONE6[_g, :6] = 1.0
    _ONESLOT[0, 31, _g, 6, 0] = 1.0
for _s in range(8):
    for _e in range(3):
        _q = _s - _e
        if 0 <= _q <= 3 and _s <= 5:
            _SEL2[_s, _e, _q] = 1.0
_B1[0, 31] = 1.0
_S6[6, 0] = 1.0
_D0 = np.array([1.0, 0.0, 0.0], np.float32)
_Q1 = np.ones(4, np.float32)


def _prep(conv1_w, conv1_b, conv2_w, conv2_b, fc1_w, fc1_b, fc2_w, fc2_b):
    w1r = conv1_w[:, 0, :, :]
    t1 = (jnp.einsum('cde,pegs->dpgsc', w1r, _SEL1)
          + jnp.einsum('dp,gs,c->dpgsc', _B1, _ONE6, conv1_b)
          + _ONESLOT).reshape(96, 1536)
    t2 = (jnp.einsum('oide,seq->dsiqo', conv2_w, _SEL2)
          + jnp.einsum('d,si,q,o->dsiqo', _D0, _S6, _Q1, conv2_b))
    t2 = t2.reshape(3, 256, 256)
    fw1 = fc1_w.reshape(128, 64, 12, 12).transpose(2, 3, 1, 0)
    fw1 = fw1.reshape(12, 768, 128)
    fw2 = jnp.zeros((128, 128), jnp.float32).at[:, :10].set(fc2_w.T)
    fb2 = jnp.zeros((1, 128), jnp.float32).at[0, :10].set(fc2_b)
    return (t1.astype(jnp.bfloat16), t2.astype(jnp.bfloat16),
            fw1.astype(jnp.bfloat16), fc1_b.reshape(1, 128),
            fw2.astype(jnp.bfloat16), fb2)


def _call(xp, args, interpret=False):
    b = xp.shape[1]
    grid = b // BT
    const = lambda *shape: pl.BlockSpec(shape, lambda i: (0,) * len(shape))
    return pl.pallas_call(
        _net_kernel,
        grid=(grid,),
        in_specs=[
            pl.BlockSpec((40, BT, 32), lambda i: (0, i, 0)),
            const(96, 1536), const(3, 256, 256), const(12, 768, 128),
            const(1, 128), const(128, 128), const(1, 128),
        ],
        out_specs=pl.BlockSpec((BT, 10), lambda i: (i, 0)),
        out_shape=jax.ShapeDtypeStruct((b, 10), jnp.float32),
        interpret=interpret,
    )(xp, *args)


def kernel(x, conv1_w, conv1_b, conv2_w, conv2_b, fc1_w, fc1_b, fc2_w, fc2_b):
    args = _prep(conv1_w, conv1_b, conv2_w, conv2_b,
                 fc1_w, fc1_b, fc2_w, fc2_b)
    xp = jnp.pad(x.reshape(x.shape[0], 28, 28), ((0, 0), (0, 12), (0, 4)))
    xp = xp.at[:, :, 31].set(1.0)
    xp = xp.transpose(1, 0, 2).astype(jnp.bfloat16)
    return _call(xp, args)


# speedup vs baseline: 2.3539x; 1.0001x over previous
import jax
import jax.numpy as jnp
import numpy as np
from jax.experimental import pallas as pl
from jax.experimental.pallas import tpu as pltpu

BT = 128


def _net_kernel(xp_ref, t1_ref, t2_ref, fw1_ref, fb1_ref, fw2_ref, fb2_ref,
                out_ref):
    x2d = xp_ref[...].reshape(40 * BT, 32)
    xcat = jnp.concatenate(
        [x2d[0:32 * BT], x2d[BT:33 * BT], x2d[2 * BT:34 * BT]], axis=1)
    x1g = []
    for g in range(6):
        h1 = jnp.dot(xcat, t1_ref[g], preferred_element_type=jnp.float32)
        x1g.append(jnp.maximum(h1.astype(jnp.bfloat16), 0))

    parts = []
    for g in range(6):
        acc = None
        for d in range(3):
            xs = x1g[g][d * BT:(d + 24) * BT]
            p = jnp.dot(xs, t2_ref[d], preferred_element_type=jnp.float32)
            acc = p if acc is None else acc + p
        h2 = jnp.maximum(acc, 0.0)
        m = jnp.max(h2.reshape(12, 2, BT, 256), axis=1)
        p1 = jnp.maximum(m[..., 0:64], m[..., 64:128])
        p2 = jnp.maximum(m[..., 128:192], m[..., 192:256])
        parts.append(jnp.concatenate([p1, p2], axis=2).astype(jnp.bfloat16))
    flat = jnp.concatenate(parts, axis=2)
    flat = flat.reshape(12 * BT, 768)

    facc = None
    for t in range(12):
        ft = jnp.dot(flat[t * BT:(t + 1) * BT], fw1_ref[t],
                     preferred_element_type=jnp.float32)
        facc = ft if facc is None else facc + ft
    f1 = jnp.maximum(facc + fb1_ref[...], 0.0).astype(jnp.bfloat16)
    f2 = jnp.dot(f1, fw2_ref[...], preferred_element_type=jnp.float32)
    out_ref[...] = (f2 + fb2_ref[...])[:, :10]


_SEL1 = np.zeros((32, 3, 6, 8), np.float32)
_SEL2 = np.zeros((8, 3, 4), np.float32)
_B1 = np.zeros((3, 32), np.float32)
_ONE6 = np.zeros((6, 8), np.float32)
_ONESLOT = np.zeros((3, 32, 6, 8, 32), np.float32)
_S6 = np.zeros((8, 32), np.float32)
for _g in range(6):
    for _s in range(6):
        for _e in range(3):
            _SEL1[4 * _g + _s + _e, _e, _g, _s] = 1.0
    _ONE6[_g, :6] = 1.0
    _ONESLOT[0, 31, _g, 6, 0] = 1.0
for _s in range(8):
    for _e in range(3):
        _q = _s - _e
        if 0 <= _q <= 3 and _s <= 5:
            _SEL2[_s, _e, _q] = 1.0
_B1[0, 31] = 1.0
_S6[6, 0] = 1.0
_D0 = np.array([1.0, 0.0, 0.0], np.float32)
_Q1 = np.ones(4, np.float32)


def _prep(conv1_w, conv1_b, conv2_w, conv2_b, fc1_w, fc1_b, fc2_w, fc2_b):
    w1r = conv1_w[:, 0, :, :]
    t1 = (jnp.einsum('cde,pegs->dpgsc', w1r, _SEL1)
          + jnp.einsum('dp,gs,c->dpgsc', _B1, _ONE6, conv1_b)
          + _ONESLOT).reshape(96, 6, 256).transpose(1, 0, 2)
    t2 = (jnp.einsum('oide,seq->dsiqo', conv2_w, _SEL2)
          + jnp.einsum('d,si,q,o->dsiqo', _D0, _S6, _Q1, conv2_b))
    t2 = t2.reshape(3, 256, 256)
    fw1 = fc1_w.reshape(128, 64, 12, 12).transpose(2, 3, 1, 0)
    fw1 = fw1.reshape(12, 768, 128)
    fw2 = jnp.zeros((128, 128), jnp.float32).at[:, :10].set(fc2_w.T)
    fb2 = jnp.zeros((1, 128), jnp.float32).at[0, :10].set(fc2_b)
    return (t1.astype(jnp.bfloat16), t2.astype(jnp.bfloat16),
            fw1.astype(jnp.bfloat16), fc1_b.reshape(1, 128),
            fw2.astype(jnp.bfloat16), fb2)


def _call(xp, args, interpret=False):
    b = xp.shape[1]
    grid = b // BT
    const = lambda *shape: pl.BlockSpec(shape, lambda i: (0,) * len(shape))
    return pl.pallas_call(
        _net_kernel,
        grid=(grid,),
        in_specs=[
            pl.BlockSpec((40, BT, 32), lambda i: (0, i, 0)),
            const(6, 96, 256), const(3, 256, 256), const(12, 768, 128),
            const(1, 128), const(128, 128), const(1, 128),
        ],
        out_specs=pl.BlockSpec((BT, 10), lambda i: (i, 0)),
        out_shape=jax.ShapeDtypeStruct((b, 10), jnp.float32),
        interpret=interpret,
    )(xp, *args)


def kernel(x, conv1_w, conv1_b, conv2_w, conv2_b, fc1_w, fc1_b, fc2_w, fc2_b):
    args = _prep(conv1_w, conv1_b, conv2_w, conv2_b,
                 fc1_w, fc1_b, fc2_w, fc2_b)
    xp = jnp.pad(x.reshape(x.shape[0], 28, 28), ((0, 0), (0, 12), (0, 4)))
    xp = xp.at[:, :, 31].set(1.0)
    xp = xp.transpose(1, 0, 2).astype(jnp.bfloat16)
    return _call(xp, args)


# speedup vs baseline: 2.4255x; 1.0304x over previous
import jax
import jax.numpy as jnp
import numpy as np
from jax.experimental import pallas as pl
from jax.experimental.pallas import tpu as pltpu

BT = 256


def _net_kernel(xp_ref, t1_ref, t2_ref, fw1_ref, fb1_ref, fw2_ref, fb2_ref,
                out_ref):
    x2d = xp_ref[...].reshape(40 * BT, 32)
    xcat = jnp.concatenate(
        [x2d[0:32 * BT], x2d[BT:33 * BT], x2d[2 * BT:34 * BT]], axis=1)
    parts = []
    for g in range(6):
        h1 = jnp.dot(xcat, t1_ref[g], preferred_element_type=jnp.float32)
        x1 = jnp.maximum(h1.astype(jnp.bfloat16), 0)
        acc = None
        for d in range(3):
            xs = x1[d * BT:(d + 24) * BT]
            p = jnp.dot(xs, t2_ref[d], preferred_element_type=jnp.float32)
            acc = p if acc is None else acc + p
        h2 = jnp.maximum(acc, 0.0)
        m = jnp.max(h2.reshape(12, 2, BT, 256), axis=1)
        p1 = jnp.maximum(m[..., 0:64], m[..., 64:128])
        p2 = jnp.maximum(m[..., 128:192], m[..., 192:256])
        parts.append(jnp.concatenate([p1, p2], axis=2).astype(jnp.bfloat16))
    flat = jnp.concatenate(parts, axis=2)
    flat = flat.reshape(12 * BT, 768)

    facc = None
    for t in range(12):
        ft = jnp.dot(flat[t * BT:(t + 1) * BT], fw1_ref[t],
                     preferred_element_type=jnp.float32)
        facc = ft if facc is None else facc + ft
    f1 = jnp.maximum(facc + fb1_ref[...], 0.0).astype(jnp.bfloat16)
    f2 = jnp.dot(f1, fw2_ref[...], preferred_element_type=jnp.float32)
    out_ref[...] = (f2 + fb2_ref[...])[:, :10]


_SEL1 = np.zeros((32, 3, 6, 8), np.float32)
_SEL2 = np.zeros((8, 3, 4), np.float32)
_B1 = np.zeros((3, 32), np.float32)
_ONE6 = np.zeros((6, 8), np.float32)
_ONESLOT = np.zeros((3, 32, 6, 8, 32), np.float32)
_S6 = np.zeros((8, 32), np.float32)
for _g in range(6):
    for _s in range(6):
        for _e in range(3):
            _SEL1[4 * _g + _s + _e, _e, _g, _s] = 1.0
    _ONE6[_g, :6] = 1.0
    _ONESLOT[0, 31, _g, 6, 0] = 1.0
for _s in range(8):
    for _e in range(3):
        _q = _s - _e
        if 0 <= _q <= 3 and _s <= 5:
            _SEL2[_s, _e, _q] = 1.0
_B1[0, 31] = 1.0
_S6[6, 0] = 1.0
_D0 = np.array([1.0, 0.0, 0.0], np.float32)
_Q1 = np.ones(4, np.float32)


def _prep(conv1_w, conv1_b, conv2_w, conv2_b, fc1_w, fc1_b, fc2_w, fc2_b):
    w1r = conv1_w[:, 0, :, :]
    t1 = (jnp.einsum('cde,pegs->dpgsc', w1r, _SEL1)
          + jnp.einsum('dp,gs,c->dpgsc', _B1, _ONE6, conv1_b)
          + _ONESLOT).reshape(96, 6, 256).transpose(1, 0, 2)
    t2 = (jnp.einsum('oide,seq->dsiqo', conv2_w, _SEL2)
          + jnp.einsum('d,si,q,o->dsiqo', _D0, _S6, _Q1, conv2_b))
    t2 = t2.reshape(3, 256, 256)
    fw1 = fc1_w.reshape(128, 64, 12, 12).transpose(2, 3, 1, 0)
    fw1 = fw1.reshape(12, 768, 128)
    fw2 = jnp.zeros((128, 128), jnp.float32).at[:, :10].set(fc2_w.T)
    fb2 = jnp.zeros((1, 128), jnp.float32).at[0, :10].set(fc2_b)
    return (t1.astype(jnp.bfloat16), t2.astype(jnp.bfloat16),
            fw1.astype(jnp.bfloat16), fc1_b.reshape(1, 128),
            fw2.astype(jnp.bfloat16), fb2)


def _call(xp, args, interpret=False):
    b = xp.shape[1]
    grid = b // BT
    const = lambda *shape: pl.BlockSpec(shape, lambda i: (0,) * len(shape))
    return pl.pallas_call(
        _net_kernel,
        grid=(grid,),
        in_specs=[
            pl.BlockSpec((40, BT, 32), lambda i: (0, i, 0)),
            const(6, 96, 256), const(3, 256, 256), const(12, 768, 128),
            const(1, 128), const(128, 128), const(1, 128),
        ],
        out_specs=pl.BlockSpec((BT, 10), lambda i: (i, 0)),
        out_shape=jax.ShapeDtypeStruct((b, 10), jnp.float32),
        interpret=interpret,
    )(xp, *args)


def kernel(x, conv1_w, conv1_b, conv2_w, conv2_b, fc1_w, fc1_b, fc2_w, fc2_b):
    args = _prep(conv1_w, conv1_b, conv2_w, conv2_b,
                 fc1_w, fc1_b, fc2_w, fc2_b)
    xp = jnp.pad(x.reshape(x.shape[0], 28, 28), ((0, 0), (0, 12), (0, 4)))
    xp = xp.at[:, :, 31].set(1.0)
    xp = xp.transpose(1, 0, 2).astype(jnp.bfloat16)
    return _call(xp, args)
